# Initial kernel scaffold; baseline (speedup 1.0000x reference)
import jax
import jax.numpy as jnp
from jax.experimental import pallas as pl


def kernel(feats, edge_dict, w_q_w, w_q_b, w_k_w, w_k_b):
    raise NotImplementedError("write your pallas kernel here")


# speedup vs baseline: 28.2611x; 28.2611x over previous
import functools

import jax
import jax.numpy as jnp
from jax import lax
from jax.experimental import pallas as pl
from jax.experimental.pallas import tpu as pltpu
from jax.experimental.pallas import tpu_sc as plsc

N = 10000
K = 16
D = 128
CHUNK = 400
NCHUNK = N // CHUNK
CPT = D // 32

_MESH = plsc.VectorSubcoreMesh(
    core_axis_name="c", subcore_axis_name="s", num_cores=2, num_subcores=16
)


def _qk_body(f_ref, wq_ref, wk_ref, bq_ref, bk_ref, o_ref):
    f = f_ref[...]
    q = jnp.sum(f * wq_ref[...], axis=1, keepdims=True) + bq_ref[...][0:1, 0:1]
    k = jnp.sum(f * wk_ref[...], axis=1, keepdims=True) + bk_ref[...][0:1, 0:1]
    o_ref[...] = jnp.concatenate([q, k], axis=1)


def _qk_project(feats, w_q_w, w_q_b, w_k_w, w_k_b):
    return pl.pallas_call(
        _qk_body,
        out_shape=jax.ShapeDtypeStruct((N, 2), jnp.float32),
    )(feats, w_q_w, w_k_w, w_q_b.reshape(1, 1), w_k_b.reshape(1, 1))


@functools.partial(
    pl.kernel,
    out_type=jax.ShapeDtypeStruct((N * K,), jnp.float32),
    mesh=_MESH,
    compiler_params=pltpu.CompilerParams(needs_layout_passes=False),
    scratch_types=[
        pltpu.VMEM((N,), jnp.float32),
        pltpu.VMEM((N,), jnp.float32),
        pltpu.VMEM((N,), jnp.int32),
        pltpu.VMEM((CHUNK * K,), jnp.int32),
        pltpu.VMEM((CHUNK * K,), jnp.float32),
        pltpu.VMEM((16,), jnp.float32),
    ],
)
def _scalar_scan(q0_hbm, k0_hbm, tgt_hbm, nbr_hbm, bias_hbm, a_hbm,
                 q_v, k_v, tgt_v, nbr_v, a_v, b_v):
    wid = lax.axis_index("s") * 2 + lax.axis_index("c")
    lane0 = lax.iota(jnp.int32, 16) == 0

    @pl.when(wid == 0)
    def _():
        pltpu.sync_copy(q0_hbm, q_v)
        pltpu.sync_copy(k0_hbm, k_v)
        pltpu.sync_copy(tgt_hbm, tgt_v)
        pltpu.sync_copy(bias_hbm, b_v)
        bqv = plsc.load_gather(b_v, [jnp.zeros((K,), jnp.int32)])
        bkv = plsc.load_gather(b_v, [jnp.ones((K,), jnp.int32)])

        def chunk_body(c, carry):
            pltpu.sync_copy(nbr_hbm.at[pl.ds(c * CHUNK * K, CHUNK * K)], nbr_v)

            def step(i, carry2):
                g = c * CHUNK + i
                tvec = plsc.load_gather(tgt_v, [jnp.full((K,), g, jnp.int32)])
                nb = nbr_v[pl.ds(i * K, K)]
                kv = plsc.load_gather(k_v, [nb])
                qv = plsc.load_gather(q_v, [nb])
                qt = plsc.load_gather(q_v, [tvec])
                e = qt * kv
                m = jnp.max(e)
                p = jnp.exp(e - m)
                s = jnp.sum(p)
                a = p / s
                a_v[pl.ds(i * K, K)] = a
                dq = jnp.sum(a * (qv - bqv))
                dk = jnp.sum(a * (kv - bkv))
                plsc.store_scatter(q_v, [tvec], qt + dq, mask=lane0)
                kt = plsc.load_gather(k_v, [tvec])
                plsc.store_scatter(k_v, [tvec], kt + dk, mask=lane0)
                return carry2

            lax.fori_loop(0, CHUNK, step, 0, unroll=False)
            pltpu.sync_copy(a_v, a_hbm.at[pl.ds(c * CHUNK * K, CHUNK * K)])
            return carry

        lax.fori_loop(0, NCHUNK, chunk_body, 0, unroll=False)


@functools.partial(
    pl.kernel,
    out_type=jax.ShapeDtypeStruct((D * N,), jnp.int32),
    mesh=_MESH,
    compiler_params=pltpu.CompilerParams(needs_layout_passes=False),
    scratch_types=[
        pltpu.VMEM((CPT * N,), jnp.float32),
        pltpu.VMEM((CPT * N,), jnp.int32),
        pltpu.VMEM((N,), jnp.int32),
        pltpu.VMEM((CHUNK * K,), jnp.int32),
        pltpu.VMEM((CHUNK * K,), jnp.float32),
    ],
)
def _vector_scan(ft_hbm, tgt_hbm, nbr_hbm, a_hbm, out_hbm,
                 f_v, o_v, tgt_v, nbr_v, a_v):
    wid = lax.axis_index("s") * 2 + lax.axis_index("c")
    base = wid * CPT * N
    lane0 = lax.iota(jnp.int32, 16) == 0
    pltpu.sync_copy(ft_hbm.at[pl.ds(base, CPT * N)], f_v)
    pltpu.sync_copy(tgt_hbm, tgt_v)

    def chunk_body(c, carry):
        pltpu.sync_copy(nbr_hbm.at[pl.ds(c * CHUNK * K, CHUNK * K)], nbr_v)
        pltpu.sync_copy(a_hbm.at[pl.ds(c * CHUNK * K, CHUNK * K)], a_v)

        def step(i, carry2):
            g = c * CHUNK + i
            gvec = jnp.full((K,), g, jnp.int32)
            tvec = plsc.load_gather(tgt_v, [gvec])
            nb = nbr_v[pl.ds(i * K, K)]
            av = a_v[pl.ds(i * K, K)]
            for col in range(CPT):
                vals = plsc.load_gather(f_v, [nb + col * N])
                u = jnp.sum(av * vals)
                fcur = plsc.load_gather(f_v, [tvec + col * N])
                nv = fcur + u
                plsc.store_scatter(f_v, [tvec + col * N], nv, mask=lane0)
                plsc.store_scatter(o_v, [gvec + col * N],
                                   nv.astype(jnp.int32), mask=lane0)
            return carry2

        lax.fori_loop(0, CHUNK, step, 0, unroll=False)
        return carry

    lax.fori_loop(0, NCHUNK, chunk_body, 0, unroll=False)
    pltpu.sync_copy(o_v, out_hbm.at[pl.ds(base, CPT * N)])


CHUNK_TC = 1000
NCHUNK_TC = N // CHUNK_TC


def _tc_scan_body(edge_sm, feats_ref, wq_ref, wk_ref, bq_sm, bk_sm,
                  out_ref, f_ref, fn_ref):
    @pl.when(pl.program_id(0) == 0)
    def _():
        f_ref[...] = feats_ref[...]

    bq = bq_sm[0]
    bk = bk_sm[0]
    wq = wq_ref[...]
    wkb = wk_ref[...]

    def step(i, carry):
        t = edge_sm[0, i, 0]
        fn = jnp.concatenate(
            [f_ref[pl.ds(edge_sm[0, i, 1 + j], 1), :] for j in range(K)],
            axis=0)
        fi = f_ref[pl.ds(t, 1), :]
        ei = jnp.sum(fi * wq, axis=1, keepdims=True) + bq
        ej = lax.dot_general(
            fn.astype(jnp.bfloat16), wkb,
            (((1,), (1,)), ((), ())),
            preferred_element_type=jnp.float32)[:, 0:1] + bk
        e = ei * ej
        m = jnp.max(e, axis=0, keepdims=True)
        p = jnp.exp(e - m)
        a = p / jnp.sum(p, axis=0, keepdims=True)
        upd = jnp.sum(a * fn, axis=0, keepdims=True)
        nr = fi + upd
        f_ref[pl.ds(t, 1), :] = nr
        out_ref[pl.ds(i, 1), :] = nr.astype(jnp.int32)
        return carry

    lax.fori_loop(0, CHUNK_TC, step, 0, unroll=False)


def _tc_scan(feats, edge, w_q_w, w_q_b, w_k_w, w_k_b):
    wk16 = jnp.tile(w_k_w.astype(jnp.bfloat16), (8, 1))
    return pl.pallas_call(
        _tc_scan_body,
        grid=(NCHUNK_TC,),
        in_specs=[
            pl.BlockSpec((1, CHUNK_TC, 17), lambda i: (i, 0, 0),
                         memory_space=pltpu.SMEM),
            pl.BlockSpec((N, D), lambda i: (0, 0)),
            pl.BlockSpec((1, D), lambda i: (0, 0)),
            pl.BlockSpec((8, D), lambda i: (0, 0)),
            pl.BlockSpec((1,), lambda i: (0,), memory_space=pltpu.SMEM),
            pl.BlockSpec((1,), lambda i: (0,), memory_space=pltpu.SMEM),
        ],
        out_specs=pl.BlockSpec((CHUNK_TC, D), lambda i: (i, 0)),
        out_shape=jax.ShapeDtypeStruct((N, D), jnp.int32),
        scratch_shapes=[
            pltpu.VMEM((N, D), jnp.float32),
            pltpu.VMEM((K, D), jnp.float32),
        ],
    )(edge.reshape(NCHUNK_TC, CHUNK_TC, 17), feats, w_q_w, wk16, w_q_b, w_k_b)


def _exact_a(feats, edge, w_q_w, w_q_b, w_k_w, w_k_b):
    def step(f, row):
        tgt = row[0]
        nbrs = row[1:]
        fi = f[tgt]
        ei = (fi @ w_q_w.T + w_q_b).reshape(1, 1)
        fn = f[nbrs]
        ej = fn @ w_k_w.T + w_k_b
        eij = ei * ej
        aij = jax.nn.softmax(eij, axis=0)
        upd = jnp.sum(aij * fn, axis=0)
        f = f.at[tgt].add(upd)
        return f, aij[:, 0]

    _, A = lax.scan(step, feats, edge)
    return A


def kernel(feats, edge_dict, w_q_w, w_q_b, w_k_w, w_k_b):
    edge = edge_dict.astype(jnp.int32)
    return _tc_scan(feats, edge, w_q_w, w_q_b, w_k_w, w_k_b)


def _kernel_sc(feats, edge_dict, w_q_w, w_q_b, w_k_w, w_k_b):
    edge = edge_dict.astype(jnp.int32)
    tgt = edge[:, 0]
    nbr_flat = edge[:, 1:].reshape(N * K)
    wk_eff = w_k_w.astype(jnp.bfloat16).astype(jnp.float32)
    qk = _qk_project(feats, w_q_w, w_q_b, wk_eff, w_k_b)
    q0 = qk[:, 0]
    k0 = qk[:, 1]
    bias_vec = jnp.concatenate(
        [w_q_b.astype(jnp.float32), w_k_b.astype(jnp.float32),
         jnp.zeros((14,), jnp.float32)]
    )
    a = _scalar_scan(q0, k0, tgt, nbr_flat, bias_vec)
    ft = feats.T.reshape(D * N)
    out_flat = _vector_scan(ft, tgt, nbr_flat, a)
    return out_flat.reshape(D, N).T
